# baseline (speedup 1.0000x reference)
import functools

import jax
import jax.numpy as jnp
from jax import lax
from jax.experimental import pallas as pl
from jax.experimental.pallas import tpu as pltpu
from jax.experimental.pallas import tpu_sc as plsc

B = 16384
DIM = 64
N_OUT = 1000

_NC = 2
_NS = 16
_NW = _NC * _NS
_BPW = B // _NW
_CHUNK = 128
_NCHUNK = _BPW // _CHUNK


def _sc_gather(table, idx3):
    mesh = plsc.VectorSubcoreMesh(core_axis_name="c", subcore_axis_name="s")

    @functools.partial(
        pl.kernel,
        mesh=mesh,
        out_type=jax.ShapeDtypeStruct((_NW, _BPW, DIM), jnp.float32),
        scratch_types=[
            pltpu.VMEM((_NCHUNK, _CHUNK), jnp.int32),
            pltpu.VMEM((_BPW, DIM), jnp.float32),
            pltpu.SemaphoreType.DMA,
        ],
        compiler_params=pltpu.CompilerParams(use_tc_tiling_on_sc=False),
    )
    def k(table_hbm, idx_hbm, out_hbm, idx_v, rows_v, sem):
        wid = lax.axis_index("s") * _NC + lax.axis_index("c")
        pltpu.sync_copy(idx_hbm.at[wid], idx_v)
        copies = [
            pltpu.async_copy(
                table_hbm.at[idx_v.at[j]],
                rows_v.at[pl.ds(j * _CHUNK, _CHUNK)],
                sem,
            )
            for j in range(_NCHUNK)
        ]
        for c in copies:
            c.wait()
        pltpu.sync_copy(rows_v, out_hbm.at[wid])

    return k(table, idx3)


def _mm_body(emb_ref, wt_ref, b_ref, out_ref):
    out_ref[...] = (
        jnp.dot(emb_ref[...], wt_ref[...], preferred_element_type=jnp.float32)
        + b_ref[...]
    )


def _tc_matmul(emb, wt, b2):
    bb = 2048
    grid = (B // bb,)
    return pl.pallas_call(
        _mm_body,
        grid=grid,
        in_specs=[
            pl.BlockSpec((bb, DIM), lambda i: (i, 0)),
            pl.BlockSpec((DIM, N_OUT), lambda i: (0, 0)),
            pl.BlockSpec((1, N_OUT), lambda i: (0, 0)),
        ],
        out_specs=pl.BlockSpec((bb, N_OUT), lambda i: (i, 0)),
        out_shape=jax.ShapeDtypeStruct((B, N_OUT), jnp.float32),
    )(emb, wt, b2)


def kernel(x, table, W, b):
    idx3 = x.astype(jnp.int32).reshape(_NW, _NCHUNK, _CHUNK)
    emb = _sc_gather(table, idx3).reshape(B, DIM)
    wt = W.T
    b2 = b.reshape(1, N_OUT)
    return _tc_matmul(emb, wt, b2)


# speedup vs baseline: 1.0001x; 1.0001x over previous
import functools

import jax
import jax.numpy as jnp
from jax import lax
from jax.experimental import pallas as pl
from jax.experimental.pallas import tpu as pltpu
from jax.experimental.pallas import tpu_sc as plsc

B = 16384
DIM = 64
N_OUT = 1000

_NC = 2
_NS = 16
_NW = _NC * _NS
_BPW = B // _NW
_CHUNK = 128
_NCHUNK = _BPW // _CHUNK


def _sc_gather_pairs(table2, idx3):
    mesh = plsc.VectorSubcoreMesh(core_axis_name="c", subcore_axis_name="s")

    @functools.partial(
        pl.kernel,
        mesh=mesh,
        out_type=jax.ShapeDtypeStruct((_NW, _BPW, 2 * DIM), jnp.float32),
        scratch_types=[
            pltpu.VMEM((_NCHUNK, _CHUNK), jnp.int32),
            pltpu.VMEM((_BPW, 2 * DIM), jnp.float32),
            pltpu.SemaphoreType.DMA,
        ],
    )
    def k(table_hbm, idx_hbm, out_hbm, idx_v, rows_v, sem):
        wid = lax.axis_index("s") * _NC + lax.axis_index("c")
        pltpu.sync_copy(idx_hbm.at[wid], idx_v)
        copies = [
            pltpu.async_copy(
                table_hbm.at[idx_v.at[j]],
                rows_v.at[pl.ds(j * _CHUNK, _CHUNK)],
                sem,
            )
            for j in range(_NCHUNK)
        ]
        for c in copies:
            c.wait()
        pltpu.sync_copy(rows_v, out_hbm.at[wid])

    return k(table2, idx3)


def _mm_body(emb2_ref, par_ref, wt_ref, b_ref, out_ref):
    lo = jnp.dot(
        emb2_ref[:, :DIM], wt_ref[...], preferred_element_type=jnp.float32
    )
    hi = jnp.dot(
        emb2_ref[:, DIM:], wt_ref[...], preferred_element_type=jnp.float32
    )
    out_ref[...] = jnp.where(par_ref[...] != 0, hi, lo) + b_ref[...]


def _tc_matmul(emb2, par, wt, b2):
    bb = 2048
    grid = (B // bb,)
    return pl.pallas_call(
        _mm_body,
        grid=grid,
        in_specs=[
            pl.BlockSpec((bb, 2 * DIM), lambda i: (i, 0)),
            pl.BlockSpec((bb, 1), lambda i: (i, 0)),
            pl.BlockSpec((DIM, N_OUT), lambda i: (0, 0)),
            pl.BlockSpec((1, N_OUT), lambda i: (0, 0)),
        ],
        out_specs=pl.BlockSpec((bb, N_OUT), lambda i: (i, 0)),
        out_shape=jax.ShapeDtypeStruct((B, N_OUT), jnp.float32),
    )(emb2, par, wt, b2)


def kernel(x, table, W, b):
    xi = x.astype(jnp.int32)
    table2 = table.reshape(table.shape[0] // 2, 2 * DIM)
    idx3 = (xi >> 1).reshape(_NW, _NCHUNK, _CHUNK)
    par = (xi & 1).reshape(B, 1)
    emb2 = _sc_gather_pairs(table2, idx3).reshape(B, 2 * DIM)
    wt = W.T
    b2 = b.reshape(1, N_OUT)
    return _tc_matmul(emb2, par, wt, b2)


# speedup vs baseline: 1.1055x; 1.1054x over previous
import functools

import jax
import jax.numpy as jnp
from jax import lax
from jax.experimental import pallas as pl
from jax.experimental.pallas import tpu as pltpu
from jax.experimental.pallas import tpu_sc as plsc

B = 16384
DIM = 64
N_OUT = 1000

_NC = 2
_NS = 16
_NW = _NC * _NS
_BPW = B // _NW
_CHUNK = 128
_NCHUNK = _BPW // _CHUNK


def _sc_gather_pairs(table2, idx3):
    mesh = plsc.VectorSubcoreMesh(core_axis_name="c", subcore_axis_name="s")

    @functools.partial(
        pl.kernel,
        mesh=mesh,
        out_type=jax.ShapeDtypeStruct((_NW, _BPW, 2 * DIM), jnp.float32),
        scratch_types=[
            pltpu.VMEM((_NCHUNK, _CHUNK), jnp.int32),
            pltpu.VMEM((_BPW, 2 * DIM), jnp.float32),
            pltpu.SemaphoreType.DMA,
        ],
    )
    def k(table_hbm, idx_hbm, out_hbm, idx_v, rows_v, sem):
        wid = lax.axis_index("s") * _NC + lax.axis_index("c")
        pltpu.sync_copy(idx_hbm.at[wid], idx_v)
        copies = [
            pltpu.async_copy(
                table_hbm.at[idx_v.at[j]],
                rows_v.at[pl.ds(j * _CHUNK, _CHUNK)],
                sem,
            )
            for j in range(_NCHUNK)
        ]
        for c in copies:
            c.wait()
        pltpu.sync_copy(rows_v, out_hbm.at[wid])

    return k(table2, idx3)


def _mm_body(emb2_ref, par_ref, w_ref, b_ref, out_ref):
    dims = (((1,), (1,)), ((), ()))
    lo = lax.dot_general(
        w_ref[...], emb2_ref[:, :DIM], dims, preferred_element_type=jnp.float32
    )
    hi = lax.dot_general(
        w_ref[...], emb2_ref[:, DIM:], dims, preferred_element_type=jnp.float32
    )
    out_ref[...] = jnp.where(par_ref[...] != 0, hi, lo) + b_ref[...]


def _tc_matmul_t(emb2, par, w, b2):
    bb = 2048
    grid = (B // bb,)
    return pl.pallas_call(
        _mm_body,
        grid=grid,
        in_specs=[
            pl.BlockSpec((bb, 2 * DIM), lambda i: (i, 0)),
            pl.BlockSpec((1, bb), lambda i: (0, i)),
            pl.BlockSpec((N_OUT, DIM), lambda i: (0, 0)),
            pl.BlockSpec((N_OUT, 1), lambda i: (0, 0)),
        ],
        out_specs=pl.BlockSpec((N_OUT, bb), lambda i: (0, i)),
        out_shape=jax.ShapeDtypeStruct((N_OUT, B), jnp.float32),
    )(emb2, par, w, b2)


def kernel(x, table, W, b):
    xi = x.astype(jnp.int32)
    table2 = table.reshape(table.shape[0] // 2, 2 * DIM)
    idx3 = (xi >> 1).reshape(_NW, _NCHUNK, _CHUNK)
    par = (xi & 1).reshape(1, B)
    emb2 = _sc_gather_pairs(table2, idx3).reshape(B, 2 * DIM)
    b2 = b.reshape(N_OUT, 1)
    out_t = _tc_matmul_t(emb2, par, W, b2)
    return out_t.T


# speedup vs baseline: 2.5747x; 2.3289x over previous
import functools

import jax
import jax.numpy as jnp
from jax import lax
from jax.experimental import pallas as pl
from jax.experimental.pallas import tpu as pltpu
from jax.experimental.pallas import tpu_sc as plsc

B = 16384
DIM = 64
N_OUT = 1000

_NC = 2
_NS = 16
_NW = _NC * _NS
_BPW = B // _NW
_CHUNK = 128
_NCHUNK = _BPW // _CHUNK

_VB = 8192
_HALF = 61 * _VB
_NPAIR = 62 * _VB


def _conv_body(ta_ref, tb_ref, out_ref):
    out_ref[:, :DIM] = ta_ref[...].T
    out_ref[:, DIM:] = tb_ref[...].T


def _tc_convert(tt):
    grid = (62,)
    return pl.pallas_call(
        _conv_body,
        grid=grid,
        in_specs=[
            pl.BlockSpec((DIM, _VB), lambda i: (0, i)),
            pl.BlockSpec((DIM, _VB), lambda i: (0, i + 61)),
        ],
        out_specs=pl.BlockSpec((_VB, 2 * DIM), lambda i: (i, 0)),
        out_shape=jax.ShapeDtypeStruct((_NPAIR, 2 * DIM), jnp.float32),
    )(tt, tt)


def _sc_gather_pairs(table2, idx3):
    mesh = plsc.VectorSubcoreMesh(core_axis_name="c", subcore_axis_name="s")

    @functools.partial(
        pl.kernel,
        mesh=mesh,
        out_type=jax.ShapeDtypeStruct((_NW, _BPW, 2 * DIM), jnp.float32),
        scratch_types=[
            pltpu.VMEM((_NCHUNK, _CHUNK), jnp.int32),
            pltpu.VMEM((_BPW, 2 * DIM), jnp.float32),
            pltpu.SemaphoreType.DMA,
        ],
    )
    def k(table_hbm, idx_hbm, out_hbm, idx_v, rows_v, sem):
        wid = lax.axis_index("s") * _NC + lax.axis_index("c")
        pltpu.sync_copy(idx_hbm.at[wid], idx_v)
        copies = [
            pltpu.async_copy(
                table_hbm.at[idx_v.at[j]],
                rows_v.at[pl.ds(j * _CHUNK, _CHUNK)],
                sem,
            )
            for j in range(_NCHUNK)
        ]
        for c in copies:
            c.wait()
        pltpu.sync_copy(rows_v, out_hbm.at[wid])

    return k(table2, idx3)


def _mm_body(emb2_ref, par_ref, w_ref, b_ref, out_ref):
    dims = (((1,), (1,)), ((), ()))
    lo = lax.dot_general(
        w_ref[...], emb2_ref[:, :DIM], dims, preferred_element_type=jnp.float32
    )
    hi = lax.dot_general(
        w_ref[...], emb2_ref[:, DIM:], dims, preferred_element_type=jnp.float32
    )
    out_ref[...] = jnp.where(par_ref[...] != 0, hi, lo) + b_ref[...]


def _tc_matmul_t(emb2, par, w, b2):
    bb = 2048
    grid = (B // bb,)
    return pl.pallas_call(
        _mm_body,
        grid=grid,
        in_specs=[
            pl.BlockSpec((bb, 2 * DIM), lambda i: (i, 0)),
            pl.BlockSpec((1, bb), lambda i: (0, i)),
            pl.BlockSpec((N_OUT, DIM), lambda i: (0, 0)),
            pl.BlockSpec((N_OUT, 1), lambda i: (0, 0)),
        ],
        out_specs=pl.BlockSpec((N_OUT, bb), lambda i: (0, i)),
        out_shape=jax.ShapeDtypeStruct((N_OUT, B), jnp.float32),
    )(emb2, par, w, b2)


def kernel(x, table, W, b):
    xi = x.astype(jnp.int32)
    table2 = _tc_convert(table.T)
    pidx = jnp.where(xi < _HALF, xi, xi - _HALF)
    idx3 = pidx.reshape(_NW, _NCHUNK, _CHUNK)
    par = (xi >= _HALF).astype(jnp.int32).reshape(1, B)
    emb2 = _sc_gather_pairs(table2, idx3).reshape(B, 2 * DIM)
    b2 = b.reshape(N_OUT, 1)
    out_t = _tc_matmul_t(emb2, par, W, b2)
    return out_t.T
